# baseline (speedup 1.0000x reference)
import dataclasses
import functools

import jax
import jax.numpy as jnp
from jax import lax
from jax.experimental import pallas as pl
from jax.experimental.pallas import tpu as pltpu
from jax.experimental.pallas import tpu_sc as plsc

_F32 = jnp.float32


def _head_rep_matrix(feat, head_ch):
    i0 = lax.broadcasted_iota(jnp.int32, (feat, feat), 0)
    i1 = lax.broadcasted_iota(jnp.int32, (feat, feat), 1)
    return (i0 // head_ch == i1 // head_ch).astype(_F32)


def _attn_tables(h, asf, adf, head_ch):
    s = _head_rep_matrix(h.shape[1], head_ch)
    as_rep = jnp.dot(h * asf, s, preferred_element_type=_F32,
                     precision=lax.Precision.HIGHEST)
    ad_rep = jnp.dot(h * adf, s, preferred_element_type=_F32,
                     precision=lax.Precision.HIGHEST)
    return as_rep, ad_rep


def _prep_body(head_ch, x_ref, w_ref, asf_ref, adf_ref, src_tab_ref, ad_ref):
    h = jnp.dot(x_ref[...], w_ref[...], preferred_element_type=_F32,
                precision=lax.Precision.HIGHEST)
    as_rep, ad_rep = _attn_tables(h, asf_ref[...], adf_ref[...], head_ch)
    src_tab_ref[...] = jnp.concatenate([h, as_rep], axis=1)
    ad_ref[...] = jnp.concatenate([ad_rep, ad_rep], axis=1)


def _prep_call(x, w, asf, adf, head_ch, rb):
    n, d = x.shape
    f = w.shape[1]
    return pl.pallas_call(
        functools.partial(_prep_body, head_ch),
        grid=(n // rb,),
        in_specs=[
            pl.BlockSpec((rb, d), lambda i: (i, 0)),
            pl.BlockSpec((d, f), lambda i: (0, 0)),
            pl.BlockSpec((1, f), lambda i: (0, 0)),
            pl.BlockSpec((1, f), lambda i: (0, 0)),
        ],
        out_specs=[
            pl.BlockSpec((rb, 2 * f), lambda i: (i, 0)),
            pl.BlockSpec((rb, 2 * f), lambda i: (i, 0)),
        ],
        out_shape=[
            jax.ShapeDtypeStruct((n, 2 * f), _F32),
            jax.ShapeDtypeStruct((n, 2 * f), _F32),
        ],
    )(x, w, asf, adf)


def _combine_self_loop(part, src_tab, ad_tab, bias):
    f = src_tab.shape[1] // 2
    h = src_tab[:, :f]
    as_rep = src_tab[:, f:]
    ad_rep = ad_tab[:, :f]
    msg = part[0, :, :f] + part[1, :, :f]
    wsum = part[0, :, f:] + part[1, :, f:]
    e_self = as_rep + ad_rep
    w_self = jnp.exp(jnp.maximum(e_self, 0.2 * e_self))
    msg = msg + h * w_self
    den = wsum + w_self + 1e-16
    return msg / den + bias


def _mid_body(part_ref, src_tab_ref, ad_ref, b1_ref, w2_ref, a2s_ref,
              a2d_ref, src_tab2_ref, ad2_ref):
    o = _combine_self_loop(part_ref[...], src_tab_ref[...], ad_ref[...],
                           b1_ref[...])
    x1 = jnp.where(o > 0, o, jnp.exp(jnp.minimum(o, 0.0)) - 1.0)
    h2 = jnp.dot(x1, w2_ref[...], preferred_element_type=_F32,
                 precision=lax.Precision.HIGHEST)
    as2, ad2 = _attn_tables(h2, a2s_ref[...], a2d_ref[...], h2.shape[1])
    src_tab2_ref[...] = jnp.concatenate([h2, as2], axis=1)
    ad2_ref[...] = jnp.concatenate([ad2, ad2], axis=1)


def _mid_call(part1, src_tab1, ad1, b1, w2, a2s, a2d, rb):
    nc, n, _ = part1.shape
    f = ad1.shape[1] // 2
    f2 = w2.shape[1]
    return pl.pallas_call(
        _mid_body,
        grid=(n // rb,),
        in_specs=[
            pl.BlockSpec((nc, rb, 2 * f), lambda i: (0, i, 0)),
            pl.BlockSpec((rb, 2 * f), lambda i: (i, 0)),
            pl.BlockSpec((rb, 2 * f), lambda i: (i, 0)),
            pl.BlockSpec((1, f), lambda i: (0, 0)),
            pl.BlockSpec((f, f2), lambda i: (0, 0)),
            pl.BlockSpec((1, f2), lambda i: (0, 0)),
            pl.BlockSpec((1, f2), lambda i: (0, 0)),
        ],
        out_specs=[
            pl.BlockSpec((rb, 2 * f2), lambda i: (i, 0)),
            pl.BlockSpec((rb, 2 * f2), lambda i: (i, 0)),
        ],
        out_shape=[
            jax.ShapeDtypeStruct((n, 2 * f2), _F32),
            jax.ShapeDtypeStruct((n, 2 * f2), _F32),
        ],
    )(part1, src_tab1, ad1, b1, w2, a2s, a2d)


def _final_body(part_ref, src_tab_ref, ad_ref, b2_ref, out_ref):
    o = _combine_self_loop(part_ref[...], src_tab_ref[...], ad_ref[...],
                           b2_ref[...])
    m = jnp.max(o, axis=1, keepdims=True)
    z = o - m
    lse = jnp.log(jnp.sum(jnp.exp(z), axis=1, keepdims=True))
    out_ref[...] = z - lse


def _final_call(part2, src_tab2, ad2, b2, rb):
    nc, n, _ = part2.shape
    f = ad2.shape[1] // 2
    return pl.pallas_call(
        _final_body,
        grid=(n // rb,),
        in_specs=[
            pl.BlockSpec((nc, rb, 2 * f), lambda i: (0, i, 0)),
            pl.BlockSpec((rb, 2 * f), lambda i: (i, 0)),
            pl.BlockSpec((rb, 2 * f), lambda i: (i, 0)),
            pl.BlockSpec((1, f), lambda i: (0, 0)),
        ],
        out_specs=pl.BlockSpec((rb, f), lambda i: (i, 0)),
        out_shape=jax.ShapeDtypeStruct((n, f), _F32),
    )(part2, src_tab2, ad2, b2)


_K = 64
_ZROWS = 40


def _edge_pass(src_tab, ad_tab, src_idx, dst_idx, ad_vec=None, kk=_K):
    n, f2 = src_tab.shape
    f = f2 // 2
    e = src_idx.shape[0]
    mesh = plsc.VectorSubcoreMesh(core_axis_name="c", subcore_axis_name="s")
    nc, ns = mesh.num_cores, mesh.num_subcores
    nw = nc * ns

    n_chunks = -(-e // (kk * nw * 2)) * nw * 2
    pad = n_chunks * kk - e
    src_idx = jnp.concatenate([src_idx, jnp.zeros((pad,), jnp.int32)])
    dst_idx = jnp.concatenate([dst_idx, jnp.full((pad,), n, jnp.int32)])
    n_acc = n + 8
    ad_tab = jnp.concatenate([ad_tab, jnp.zeros((8, f2), _F32)], axis=0)
    if ad_vec is not None:
        ad_vec = jnp.concatenate([ad_vec, jnp.zeros((8,), _F32)])
    cpw = n_chunks // nw
    pairs = cpw // 2

    per_sub = -(-n // (ns * _ZROWS)) * _ZROWS
    last_rows = n - (ns - 1) * per_sub
    assert last_rows > 0 and last_rows % _ZROWS == 0
    full_z = per_sub // _ZROWS
    last_z = last_rows // _ZROWS
    assert _ZROWS <= kk and (kk % 8 == 0)

    scratch = [
        pltpu.VMEM_SHARED((n_acc, f2), _F32),
        [pltpu.VMEM((kk,), jnp.int32)] * 2,
        [pltpu.VMEM((kk,), jnp.int32)] * 2,
        [pltpu.VMEM((kk, f2), _F32)] * 2,
        [pltpu.VMEM((kk, f2), _F32)] * 2,
        pltpu.VMEM((kk, f2), _F32),
        [pltpu.SemaphoreType.DMA] * 2,
        [pltpu.SemaphoreType.DMA] * 2,
    ]
    if ad_vec is not None:
        scratch[4] = pltpu.VMEM((n_acc,), _F32)
        scratch.append(pltpu.VMEM((kk,), _F32))

    def body(src_tab_hbm, ad_tab_hbm, sidx_hbm, didx_hbm, part_hbm,
             acc, idx_s, idx_d, g_src, g_ad, outb, sem_s, sem_a, wbuf=None):
        cid = lax.axis_index("c")
        sid = lax.axis_index("s")
        wid = sid * nc + cid

        @pl.loop(0, kk)
        def _(r):
            for cc in range(f2 // 16):
                outb[r, pl.ds(16 * cc, 16)] = jnp.zeros((16,), _F32)

        row0 = sid * per_sub
        zsrc = outb.at[pl.ds(0, _ZROWS)]
        for t in range(full_z):
            if t < last_z:
                pltpu.sync_copy(zsrc,
                                acc.at[pl.ds(row0 + t * _ZROWS, _ZROWS)])
            else:
                @pl.when(sid < ns - 1)
                def _():
                    pltpu.sync_copy(zsrc,
                                    acc.at[pl.ds(row0 + t * _ZROWS, _ZROWS)])

        if ad_vec is not None:
            pltpu.sync_copy(ad_tab_hbm, g_ad)
        plsc.subcore_barrier()

        def start_load(t, b):
            base = (wid + t * nw) * kk
            pltpu.sync_copy(sidx_hbm.at[pl.ds(base, kk)], idx_s[b])
            pltpu.sync_copy(didx_hbm.at[pl.ds(base, kk)], idx_d[b])
            pltpu.make_async_copy(src_tab_hbm.at[idx_s[b]], g_src[b],
                                  sem_s[b]).start()
            if ad_vec is None:
                pltpu.make_async_copy(ad_tab_hbm.at[idx_d[b]], g_ad[b],
                                      sem_a[b]).start()

        start_load(0, 0)
        start_load(1, 1)

        iota16 = lax.broadcasted_iota(jnp.int32, (16,), 0)

        @pl.loop(0, pairs)
        def _(jj):
            for b in range(2):
                t = jj * 2 + b
                pltpu.make_async_copy(src_tab_hbm.at[idx_s[b]], g_src[b],
                                      sem_s[b]).wait()
                if ad_vec is None:
                    pltpu.make_async_copy(ad_tab_hbm.at[idx_d[b]], g_ad[b],
                                          sem_a[b]).wait()

                    @pl.loop(0, kk)
                    def _(r):
                        for cc in range(f // 16):
                            a = g_src[b][r, pl.ds(f + 16 * cc, 16)]
                            bb = g_ad[b][r, pl.ds(16 * cc, 16)]
                            ee = a + bb
                            w = jnp.exp(jnp.maximum(ee, 0.2 * ee))
                            h = g_src[b][r, pl.ds(16 * cc, 16)]
                            outb[r, pl.ds(16 * cc, 16)] = h * w
                            outb[r, pl.ds(f + 16 * cc, 16)] = w
                else:
                    for g in range(kk // 16):
                        rvec = iota16 + (16 * g)
                        d16 = idx_d[b][pl.ds(16 * g, 16)]
                        ad16 = plsc.load_gather(g_ad, [d16])
                        as16 = plsc.load_gather(
                            g_src[b], [rvec, iota16 * 0 + f])
                        ee = as16 + ad16
                        w16 = jnp.exp(jnp.maximum(ee, 0.2 * ee))
                        for rr in range(16):
                            r = 16 * g + rr
                            ws = w16.at[iota16 * 0 + rr].get(
                                mode=lax.GatherScatterMode.PROMISE_IN_BOUNDS)
                            for cc in range(f // 16):
                                h = g_src[b][r, pl.ds(16 * cc, 16)]
                                outb[r, pl.ds(16 * cc, 16)] = h * ws
                                outb[r, pl.ds(f + 16 * cc, 16)] = ws

                pltpu.sync_copy(outb, acc.at[idx_d[b]], add=True)

                @pl.when(t + 2 < cpw)
                def _():
                    start_load(t + 2, b)

        plsc.subcore_barrier()

        @pl.when(sid < ns - 1)
        def _():
            pltpu.sync_copy(
                acc.at[pl.ds(row0, per_sub)],
                part_hbm.at[cid].at[pl.ds(row0, per_sub)])

        @pl.when(sid == ns - 1)
        def _():
            pltpu.sync_copy(
                acc.at[pl.ds(row0, last_rows)],
                part_hbm.at[cid].at[pl.ds(row0, last_rows)])

    cp = pltpu.CompilerParams()
    if "needs_layout_passes" in pltpu.CompilerParams.__dataclass_fields__:
        cp = dataclasses.replace(cp, needs_layout_passes=False)
    sc_kernel = functools.partial(
        pl.kernel,
        out_type=jax.ShapeDtypeStruct((nc, n, f2), _F32),
        mesh=mesh,
        scratch_types=scratch,
        compiler_params=cp,
    )(body)

    ad_in = ad_tab if ad_vec is None else ad_vec
    return sc_kernel(src_tab, ad_in, src_idx, dst_idx)


def kernel(nodes, edges, W1, a_src1, a_dst1, b1, W2, a_src2, a_dst2, b2):
    rb = 1000
    src_idx = edges[0]
    dst_idx = edges[1]
    asf1 = a_src1.reshape(1, -1)
    adf1 = a_dst1.reshape(1, -1)
    asf2 = a_src2.reshape(1, -1)
    adf2 = a_dst2.reshape(1, -1)
    b1r = b1.reshape(1, -1)
    b2r = b2.reshape(1, -1)

    src_tab1, ad1 = _prep_call(nodes, W1, asf1, adf1, a_src1.shape[1], rb)
    part1 = _edge_pass(src_tab1, ad1, src_idx, dst_idx)
    src_tab2, ad2 = _mid_call(part1, src_tab1, ad1, b1r, W2, asf2, adf2, rb)
    part2 = _edge_pass(src_tab2, ad2, src_idx, dst_idx, ad_vec=ad2[:, 0],
                       kk=96)
    return _final_call(part2, src_tab2, ad2, b2r, rb)


# speedup vs baseline: 1.2991x; 1.2991x over previous
import dataclasses
import functools

import jax
import jax.numpy as jnp
from jax import lax
from jax.experimental import pallas as pl
from jax.experimental.pallas import tpu as pltpu
from jax.experimental.pallas import tpu_sc as plsc

_F32 = jnp.float32


def _head_rep_matrix(feat, head_ch):
    i0 = lax.broadcasted_iota(jnp.int32, (feat, feat), 0)
    i1 = lax.broadcasted_iota(jnp.int32, (feat, feat), 1)
    return (i0 // head_ch == i1 // head_ch).astype(_F32)


def _attn_tables(h, asf, adf, head_ch):
    s = _head_rep_matrix(h.shape[1], head_ch)
    as_rep = jnp.dot(h * asf, s, preferred_element_type=_F32,
                     precision=lax.Precision.HIGHEST)
    ad_rep = jnp.dot(h * adf, s, preferred_element_type=_F32,
                     precision=lax.Precision.HIGHEST)
    return as_rep, ad_rep


def _prep_body(head_ch, x_ref, w_ref, asf_ref, adf_ref, src_tab_ref, ad_ref):
    h = jnp.dot(x_ref[...], w_ref[...], preferred_element_type=_F32,
                precision=lax.Precision.HIGHEST)
    as_rep, ad_rep = _attn_tables(h, asf_ref[...], adf_ref[...], head_ch)
    src_tab_ref[...] = jnp.concatenate([h, as_rep], axis=1)
    ad_ref[...] = jnp.concatenate([ad_rep, ad_rep], axis=1)


def _prep_call(x, w, asf, adf, head_ch, rb):
    n, d = x.shape
    f = w.shape[1]
    return pl.pallas_call(
        functools.partial(_prep_body, head_ch),
        grid=(n // rb,),
        in_specs=[
            pl.BlockSpec((rb, d), lambda i: (i, 0)),
            pl.BlockSpec((d, f), lambda i: (0, 0)),
            pl.BlockSpec((1, f), lambda i: (0, 0)),
            pl.BlockSpec((1, f), lambda i: (0, 0)),
        ],
        out_specs=[
            pl.BlockSpec((rb, 2 * f), lambda i: (i, 0)),
            pl.BlockSpec((rb, 2 * f), lambda i: (i, 0)),
        ],
        out_shape=[
            jax.ShapeDtypeStruct((n, 2 * f), _F32),
            jax.ShapeDtypeStruct((n, 2 * f), _F32),
        ],
    )(x, w, asf, adf)


def _combine_self_loop(part, src_tab, ad_tab, bias):
    f = src_tab.shape[1] // 2
    h = src_tab[:, :f]
    as_rep = src_tab[:, f:]
    ad_rep = ad_tab[:, :f]
    msg = part[0, :, :f] + part[1, :, :f]
    wsum = part[0, :, f:] + part[1, :, f:]
    e_self = as_rep + ad_rep
    w_self = jnp.exp(jnp.maximum(e_self, 0.2 * e_self))
    msg = msg + h * w_self
    den = wsum + w_self + 1e-16
    return msg / den + bias


def _mid_body(part_ref, src_tab_ref, ad_ref, b1_ref, w2_ref, a2s_ref,
              a2d_ref, src_tab2_ref, ad2_ref):
    o = _combine_self_loop(part_ref[...], src_tab_ref[...], ad_ref[...],
                           b1_ref[...])
    x1 = jnp.where(o > 0, o, jnp.exp(jnp.minimum(o, 0.0)) - 1.0)
    h2 = jnp.dot(x1, w2_ref[...], preferred_element_type=_F32,
                 precision=lax.Precision.HIGHEST)
    as2, ad2 = _attn_tables(h2, a2s_ref[...], a2d_ref[...], h2.shape[1])
    src_tab2_ref[...] = jnp.concatenate([h2, as2], axis=1)
    ad2_ref[...] = jnp.concatenate([ad2, ad2], axis=1)


def _mid_call(part1, src_tab1, ad1, b1, w2, a2s, a2d, rb):
    nc, n, _ = part1.shape
    f = ad1.shape[1] // 2
    f2 = w2.shape[1]
    return pl.pallas_call(
        _mid_body,
        grid=(n // rb,),
        in_specs=[
            pl.BlockSpec((nc, rb, 2 * f), lambda i: (0, i, 0)),
            pl.BlockSpec((rb, 2 * f), lambda i: (i, 0)),
            pl.BlockSpec((rb, 2 * f), lambda i: (i, 0)),
            pl.BlockSpec((1, f), lambda i: (0, 0)),
            pl.BlockSpec((f, f2), lambda i: (0, 0)),
            pl.BlockSpec((1, f2), lambda i: (0, 0)),
            pl.BlockSpec((1, f2), lambda i: (0, 0)),
        ],
        out_specs=[
            pl.BlockSpec((rb, 2 * f2), lambda i: (i, 0)),
            pl.BlockSpec((rb, 2 * f2), lambda i: (i, 0)),
        ],
        out_shape=[
            jax.ShapeDtypeStruct((n, 2 * f2), _F32),
            jax.ShapeDtypeStruct((n, 2 * f2), _F32),
        ],
    )(part1, src_tab1, ad1, b1, w2, a2s, a2d)


def _final_body(part_ref, src_tab_ref, ad_ref, b2_ref, out_ref):
    o = _combine_self_loop(part_ref[...], src_tab_ref[...], ad_ref[...],
                           b2_ref[...])
    m = jnp.max(o, axis=1, keepdims=True)
    z = o - m
    lse = jnp.log(jnp.sum(jnp.exp(z), axis=1, keepdims=True))
    out_ref[...] = z - lse


def _final_call(part2, src_tab2, ad2, b2, rb):
    nc, n, _ = part2.shape
    f = ad2.shape[1] // 2
    return pl.pallas_call(
        _final_body,
        grid=(n // rb,),
        in_specs=[
            pl.BlockSpec((nc, rb, 2 * f), lambda i: (0, i, 0)),
            pl.BlockSpec((rb, 2 * f), lambda i: (i, 0)),
            pl.BlockSpec((rb, 2 * f), lambda i: (i, 0)),
            pl.BlockSpec((1, f), lambda i: (0, 0)),
        ],
        out_specs=pl.BlockSpec((rb, f), lambda i: (i, 0)),
        out_shape=jax.ShapeDtypeStruct((n, f), _F32),
    )(part2, src_tab2, ad2, b2)


_K = 64
_ZROWS = 40


def _edge_pass(src_tab, ad_tab, src_idx, dst_idx, ad_vec=None, kk=_K):
    n, f2 = src_tab.shape
    f = f2 // 2
    e = src_idx.shape[0]
    mesh = plsc.VectorSubcoreMesh(core_axis_name="c", subcore_axis_name="s")
    nc, ns = mesh.num_cores, mesh.num_subcores
    nw = nc * ns

    n_chunks = -(-e // (kk * nw * 2)) * nw * 2
    pad = n_chunks * kk - e
    src_idx = jnp.concatenate([src_idx, jnp.zeros((pad,), jnp.int32)])
    dst_idx = jnp.concatenate([dst_idx, jnp.full((pad,), n, jnp.int32)])
    n_acc = n + 8
    ad_tab = jnp.concatenate([ad_tab, jnp.zeros((8, f2), _F32)], axis=0)
    if ad_vec is not None:
        ad_vec = jnp.concatenate([ad_vec, jnp.zeros((8,), _F32)])
    cpw = n_chunks // nw
    pairs = cpw // 2

    per_sub = -(-n // (ns * _ZROWS)) * _ZROWS
    last_rows = n - (ns - 1) * per_sub
    assert last_rows > 0 and last_rows % _ZROWS == 0
    full_z = per_sub // _ZROWS
    last_z = last_rows // _ZROWS
    assert _ZROWS <= kk and (kk % 8 == 0)

    scratch = [
        pltpu.VMEM_SHARED((n_acc, f2), _F32),
        [pltpu.VMEM((kk,), jnp.int32)] * 2,
        [pltpu.VMEM((kk,), jnp.int32)] * 2,
        [pltpu.VMEM((kk, f2), _F32)] * 2,
        [pltpu.VMEM((kk, f2), _F32)] * 2,
        pltpu.VMEM((kk, f2), _F32),
        [pltpu.SemaphoreType.DMA] * 2,
        [pltpu.SemaphoreType.DMA] * 2,
        [pltpu.SemaphoreType.DMA] * 2,
    ]
    if ad_vec is not None:
        scratch[4] = pltpu.VMEM((n_acc,), _F32)
        scratch.append(pltpu.VMEM((kk,), _F32))

    def body(src_tab_hbm, ad_tab_hbm, sidx_hbm, didx_hbm, part_hbm,
             acc, idx_s, idx_d, g_src, g_ad, outb, sem_s, sem_a, sem_i,
             wbuf=None):
        cid = lax.axis_index("c")
        sid = lax.axis_index("s")
        wid = sid * nc + cid

        @pl.loop(0, kk)
        def _(r):
            for cc in range(f2 // 16):
                outb[r, pl.ds(16 * cc, 16)] = jnp.zeros((16,), _F32)

        row0 = sid * per_sub
        zsrc = outb.at[pl.ds(0, _ZROWS)]
        for t in range(full_z):
            if t < last_z:
                pltpu.sync_copy(zsrc,
                                acc.at[pl.ds(row0 + t * _ZROWS, _ZROWS)])
            else:
                @pl.when(sid < ns - 1)
                def _():
                    pltpu.sync_copy(zsrc,
                                    acc.at[pl.ds(row0 + t * _ZROWS, _ZROWS)])

        if ad_vec is not None:
            pltpu.sync_copy(ad_tab_hbm, g_ad)
        plsc.subcore_barrier()

        def start_load(t, b):
            base = (wid + t * nw) * kk
            pltpu.sync_copy(sidx_hbm.at[pl.ds(base, kk)], idx_s[b])
            pltpu.sync_copy(didx_hbm.at[pl.ds(base, kk)], idx_d[b])
            pltpu.make_async_copy(src_tab_hbm.at[idx_s[b]], g_src[b],
                                  sem_s[b]).start()
            if ad_vec is None:
                pltpu.make_async_copy(ad_tab_hbm.at[idx_d[b]], g_ad[b],
                                      sem_a[b]).start()

        start_load(0, 0)
        start_load(1, 1)

        iota16 = lax.broadcasted_iota(jnp.int32, (16,), 0)

        @pl.loop(0, pairs)
        def _(jj):
            for b in range(2):
                t = jj * 2 + b
                pltpu.make_async_copy(src_tab_hbm.at[idx_s[b]], g_src[b],
                                      sem_s[b]).wait()

                @pl.when(t + 2 < cpw)
                def _():
                    base2 = (wid + (t + 2) * nw) * kk
                    pltpu.make_async_copy(sidx_hbm.at[pl.ds(base2, kk)],
                                          idx_s[b], sem_i[b]).start()

                if ad_vec is None:
                    pltpu.make_async_copy(ad_tab_hbm.at[idx_d[b]], g_ad[b],
                                          sem_a[b]).wait()

                    @pl.loop(0, kk)
                    def _(r):
                        for cc in range(f // 16):
                            a = g_src[b][r, pl.ds(f + 16 * cc, 16)]
                            bb = g_ad[b][r, pl.ds(16 * cc, 16)]
                            ee = a + bb
                            w = jnp.exp(jnp.maximum(ee, 0.2 * ee))
                            h = g_src[b][r, pl.ds(16 * cc, 16)]
                            outb[r, pl.ds(16 * cc, 16)] = h * w
                            outb[r, pl.ds(f + 16 * cc, 16)] = w
                else:
                    @pl.when(t >= 2)
                    def _():
                        base0 = (wid + t * nw) * kk
                        pltpu.make_async_copy(
                            didx_hbm.at[pl.ds(base0, kk)], idx_d[b],
                            sem_a[b]).wait()

                    for g in range(kk // 16):
                        rvec = iota16 + (16 * g)
                        d16 = idx_d[b][pl.ds(16 * g, 16)]
                        ad16 = plsc.load_gather(g_ad, [d16])
                        as16 = plsc.load_gather(
                            g_src[b], [rvec, iota16 * 0 + f])
                        ee = as16 + ad16
                        w16 = jnp.exp(jnp.maximum(ee, 0.2 * ee))
                        for rr in range(16):
                            r = 16 * g + rr
                            ws = w16.at[iota16 * 0 + rr].get(
                                mode=lax.GatherScatterMode.PROMISE_IN_BOUNDS)
                            for cc in range(f // 16):
                                h = g_src[b][r, pl.ds(16 * cc, 16)]
                                outb[r, pl.ds(16 * cc, 16)] = h * ws
                                outb[r, pl.ds(f + 16 * cc, 16)] = ws

                pltpu.sync_copy(outb, acc.at[idx_d[b]], add=True)

                @pl.when(t + 2 < cpw)
                def _():
                    base2 = (wid + (t + 2) * nw) * kk
                    if ad_vec is None:
                        pltpu.sync_copy(didx_hbm.at[pl.ds(base2, kk)],
                                        idx_d[b])
                    else:
                        pltpu.make_async_copy(
                            didx_hbm.at[pl.ds(base2, kk)], idx_d[b],
                            sem_a[b]).start()
                    pltpu.make_async_copy(sidx_hbm.at[pl.ds(base2, kk)],
                                          idx_s[b], sem_i[b]).wait()
                    pltpu.make_async_copy(src_tab_hbm.at[idx_s[b]],
                                          g_src[b], sem_s[b]).start()
                    if ad_vec is None:
                        pltpu.make_async_copy(ad_tab_hbm.at[idx_d[b]],
                                              g_ad[b], sem_a[b]).start()

        plsc.subcore_barrier()

        @pl.when(sid < ns - 1)
        def _():
            pltpu.sync_copy(
                acc.at[pl.ds(row0, per_sub)],
                part_hbm.at[cid].at[pl.ds(row0, per_sub)])

        @pl.when(sid == ns - 1)
        def _():
            pltpu.sync_copy(
                acc.at[pl.ds(row0, last_rows)],
                part_hbm.at[cid].at[pl.ds(row0, last_rows)])

    cp = pltpu.CompilerParams()
    if "needs_layout_passes" in pltpu.CompilerParams.__dataclass_fields__:
        cp = dataclasses.replace(cp, needs_layout_passes=False)
    sc_kernel = functools.partial(
        pl.kernel,
        out_type=jax.ShapeDtypeStruct((nc, n, f2), _F32),
        mesh=mesh,
        scratch_types=scratch,
        compiler_params=cp,
    )(body)

    ad_in = ad_tab if ad_vec is None else ad_vec
    return sc_kernel(src_tab, ad_in, src_idx, dst_idx)


def kernel(nodes, edges, W1, a_src1, a_dst1, b1, W2, a_src2, a_dst2, b2):
    rb = 1000
    src_idx = edges[0]
    dst_idx = edges[1]
    asf1 = a_src1.reshape(1, -1)
    adf1 = a_dst1.reshape(1, -1)
    asf2 = a_src2.reshape(1, -1)
    adf2 = a_dst2.reshape(1, -1)
    b1r = b1.reshape(1, -1)
    b2r = b2.reshape(1, -1)

    src_tab1, ad1 = _prep_call(nodes, W1, asf1, adf1, a_src1.shape[1], rb)
    part1 = _edge_pass(src_tab1, ad1, src_idx, dst_idx)
    src_tab2, ad2 = _mid_call(part1, src_tab1, ad1, b1r, W2, asf2, adf2, rb)
    part2 = _edge_pass(src_tab2, ad2, src_idx, dst_idx, ad_vec=ad2[:, 0])
    return _final_call(part2, src_tab2, ad2, b2r, rb)
